# baseline (speedup 1.0000x reference)
import jax
import jax.numpy as jnp
from jax.experimental import pallas as pl
from jax.experimental.pallas import tpu as pltpu

N = 2048
BLK = 512
CH = 128
B = 128


def _attrT_kernel(w_ref, x_ref, b_ref, out_ref):
    acc = jax.lax.dot_general(w_ref[...], x_ref[...], (((0,), (1,)), ((), ())),
                              preferred_element_type=jnp.float32)
    out_ref[...] = jax.nn.sigmoid(acc + b_ref[...])


def _nrm2_kernel(t_ref, o_ref):
    a = t_ref[...]
    aa = a * a
    acc = None
    for s in range(0, N, 384):
        e = min(s + 384, N)
        w = aa[s:e].reshape(-1, 8, aa.shape[1]).sum(axis=0)
        p = w.sum(axis=0, keepdims=True)
        acc = p if acc is None else acc + p
    o_ref[...] = acc


def _gram_kernel(a_ref, b_ref, o_ref):
    ga = a_ref[...].astype(jnp.bfloat16)
    gb = b_ref[...].astype(jnp.bfloat16)
    o_ref[...] = jax.lax.dot_general(ga, gb, (((0,), (0,)), ((), ())),
                                     preferred_element_type=jnp.float32)


def _maxval_kernel(g_ref, n_ref, nT_ref, o_ref):
    denom = jnp.maximum(nT_ref[...] * n_ref[...], 1e-8)
    adjT = g_ref[...] / denom
    o_ref[...] = jnp.max(adjT, axis=0, keepdims=True)


def _anT_kernel(g_ref, n_ref, nT_ref, mv_ref, o_ref):
    denom = jnp.maximum(nT_ref[...] * n_ref[...], 1e-8)
    o_ref[...] = g_ref[...] / (denom * mv_ref[...])


def _roll0(a, shift):
    s = shift % a.shape[0]
    if s == 0:
        return a
    return jax.lax.concatenate([a[-s:], a[:-s]], 0)


def _cmp_first(av, ai, bv, bi):
    return (av > bv) | ((av == bv) & (ai < bi))


def _ce_pass(v, ix, j, dir_desc):
    rows = v.shape[0]
    it = jax.lax.broadcasted_iota(jnp.int32, (rows, 1), 0)
    upper = (it & j) != 0
    vd, vu = _roll0(v, j), _roll0(v, -j)
    id_, iu = _roll0(ix, j), _roll0(ix, -j)
    pv = jnp.where(upper, vd, vu)
    pi = jnp.where(upper, id_, iu)
    a_first = _cmp_first(v, ix, pv, pi)
    keep_early = upper == dir_desc
    sel_self = a_first == keep_early
    return jnp.where(sel_self, v, pv), jnp.where(sel_self, ix, pi)


def _local_sort(v, ix, hi_dir):
    rows = v.shape[0]
    it = jax.lax.broadcasted_iota(jnp.int32, (rows, 1), 0)
    k = 2
    while k <= rows:
        if k < rows:
            dir_desc = (it & k) != 0
        else:
            dir_desc = jnp.broadcast_to(hi_dir, (rows, 1))
        j = k // 2
        while j >= 1:
            v, ix = _ce_pass(v, ix, j, dir_desc)
            j //= 2
        k *= 2
    return v, ix


def _merge_tail(v, ix, dir_desc):
    j = CH // 2
    while j >= 1:
        v, ix = _ce_pass(v, ix, j, dir_desc)
        j //= 2
    return v, ix


def _sort_strip_kernel(a_ref, val_ref, idx_ref, v_ref, i_ref):
    nch = N // CH

    def local_body(c, _):
        base = c * CH
        v = a_ref[pl.ds(base, CH), :]
        ix = (jax.lax.broadcasted_iota(jnp.int32, (CH, 1), 0) + base
              ).astype(jnp.float32)
        ix = jnp.broadcast_to(ix, (CH, B))
        hi = ((c & 1) != 0)
        sv, si = _local_sort(v, ix, hi)
        v_ref[pl.ds(base, CH), :] = sv
        i_ref[pl.ds(base, CH), :] = si
        return 0
    jax.lax.fori_loop(0, nch, local_body, 0)

    k = 2 * CH
    while k <= N:
        j = k // 2
        while j >= CH:
            def pair_body(t, _, j=j, k=k):
                base = t * CH
                r0 = 2 * (base & ~(j - 1)) + (base & (j - 1))
                r1 = r0 + j
                av = v_ref[pl.ds(r0, CH), :]
                ai = i_ref[pl.ds(r0, CH), :]
                bv = v_ref[pl.ds(r1, CH), :]
                bi = i_ref[pl.ds(r1, CH), :]
                a_first = _cmp_first(av, ai, bv, bi)
                ev = jnp.where(a_first, av, bv)
                ei = jnp.where(a_first, ai, bi)
                lv = jnp.where(a_first, bv, av)
                li = jnp.where(a_first, bi, ai)
                ke = (r0 & k) == 0
                v_ref[pl.ds(r0, CH), :] = jnp.where(ke, ev, lv)
                i_ref[pl.ds(r0, CH), :] = jnp.where(ke, ei, li)
                v_ref[pl.ds(r1, CH), :] = jnp.where(ke, lv, ev)
                i_ref[pl.ds(r1, CH), :] = jnp.where(ke, li, ei)
                return 0
            jax.lax.fori_loop(0, N // (2 * CH), pair_body, 0)
            j //= 2

        def tail_body(c, _, k=k):
            base = c * CH
            v = v_ref[pl.ds(base, CH), :]
            ix = i_ref[pl.ds(base, CH), :]
            dd = (base & k) != 0
            sv, si = _merge_tail(v, ix, jnp.full((CH, 1), True) & dd)
            v_ref[pl.ds(base, CH), :] = sv
            i_ref[pl.ds(base, CH), :] = si
            return 0
        jax.lax.fori_loop(0, nch, tail_body, 0)
        k *= 2

    val_ref[...] = v_ref[...]
    idx_ref[...] = i_ref[...].astype(jnp.int32)


def _pipeline(x, W, b):
    nb = N // BLK
    attrT = pl.pallas_call(
        _attrT_kernel, grid=(nb, nb),
        in_specs=[pl.BlockSpec((N, BLK), lambda j, i: (0, j)),
                  pl.BlockSpec((BLK, N), lambda j, i: (i, 0)),
                  pl.BlockSpec((BLK, 1), lambda j, i: (j, 0))],
        out_specs=pl.BlockSpec((BLK, BLK), lambda j, i: (j, i)),
        out_shape=jax.ShapeDtypeStruct((N, N), jnp.float32),
    )(W, x, b.reshape(N, 1))

    nrm2 = pl.pallas_call(
        _nrm2_kernel, grid=(nb,),
        in_specs=[pl.BlockSpec((N, BLK), lambda i: (0, i))],
        out_specs=pl.BlockSpec((1, BLK), lambda i: (0, i)),
        out_shape=jax.ShapeDtypeStruct((1, N), jnp.float32),
    )(attrT)

    g = pl.pallas_call(
        _gram_kernel, grid=(nb, nb),
        in_specs=[pl.BlockSpec((N, BLK), lambda i, j: (0, i)),
                  pl.BlockSpec((N, BLK), lambda i, j: (0, j))],
        out_specs=pl.BlockSpec((BLK, BLK), lambda i, j: (i, j)),
        out_shape=jax.ShapeDtypeStruct((N, N), jnp.float32),
    )(attrT, attrT)

    norms = jnp.sqrt(nrm2)
    norms_col = norms.reshape(N, 1)
    maxval = pl.pallas_call(
        _maxval_kernel, grid=(nb,),
        in_specs=[pl.BlockSpec((N, BLK), lambda i: (0, i)),
                  pl.BlockSpec((1, BLK), lambda i: (0, i)),
                  pl.BlockSpec((N, 1), lambda i: (0, 0))],
        out_specs=pl.BlockSpec((1, BLK), lambda i: (0, i)),
        out_shape=jax.ShapeDtypeStruct((1, N), jnp.float32),
    )(g, norms, norms_col)

    anT = pl.pallas_call(
        _anT_kernel, grid=(nb,),
        in_specs=[pl.BlockSpec((N, BLK), lambda i: (0, i)),
                  pl.BlockSpec((1, BLK), lambda i: (0, i)),
                  pl.BlockSpec((N, 1), lambda i: (0, 0)),
                  pl.BlockSpec((1, BLK), lambda i: (0, i))],
        out_specs=pl.BlockSpec((N, BLK), lambda i: (0, i)),
        out_shape=jax.ShapeDtypeStruct((N, N), jnp.float32),
    )(g, norms, norms_col, maxval)

    valsT, idxT = pl.pallas_call(
        _sort_strip_kernel, grid=(N // B,),
        in_specs=[pl.BlockSpec((N, B), lambda i: (0, i))],
        out_specs=[pl.BlockSpec((N, B), lambda i: (0, i)),
                   pl.BlockSpec((N, B), lambda i: (0, i))],
        out_shape=[jax.ShapeDtypeStruct((N, N), jnp.float32),
                   jax.ShapeDtypeStruct((N, N), jnp.int32)],
        scratch_shapes=[pltpu.VMEM((N, B), jnp.float32),
                        pltpu.VMEM((N, B), jnp.float32)],
    )(anT)
    return anT, valsT, idxT


def kernel(x, W, b):
    anT, valsT, idxT = _pipeline(x, W, b)
    values = valsT.T.reshape(-1)
    src = jnp.repeat(jnp.arange(N, dtype=jnp.int32), N)
    edge_index = jnp.stack([src, idxT.T.reshape(-1)])
    return (values, edge_index, anT.T)
